# Initial kernel scaffold; baseline (speedup 1.0000x reference)
import jax
import jax.numpy as jnp
from jax.experimental import pallas as pl


def kernel(images, w6_pad, b6_pad, w7_pad, b7_pad, wcb_pad, bcb_pad, wt_cat, bt_cat, w1_bd_pad, b1_cat_pad):
    raise NotImplementedError("write your pallas kernel here")


# speedup vs baseline: 3.0179x; 3.0179x over previous
import jax
import jax.numpy as jnp
from jax.experimental import pallas as pl
from jax.experimental.pallas import tpu as pltpu

_NUM_CLASSES = 4
_ROIS_PER_IMG = 8
_C = 32
_BOX_OUT = _NUM_CLASSES + 4 * _NUM_CLASSES


def _pool_kernel(x_ref, out_ref, *, hblk, wblk, inv):
    c = pl.program_id(1)
    x = x_ref[0, 0]
    h, w = x.shape
    hi = jax.lax.broadcasted_iota(jnp.int32, (8, h), 1)
    ii = jax.lax.broadcasted_iota(jnp.int32, (8, h), 0)
    s_left = (hi // hblk == ii).astype(jnp.float32)
    wi = jax.lax.broadcasted_iota(jnp.int32, (w, 8), 0)
    ji = jax.lax.broadcasted_iota(jnp.int32, (w, 8), 1)
    s_right = (wi // wblk == ji).astype(jnp.float32)
    rows = jnp.dot(s_left, x, preferred_element_type=jnp.float32)
    part = jnp.dot(rows, s_right, preferred_element_type=jnp.float32) * inv

    @pl.when(c == 0)
    def _init():
        out_ref[0] = part

    @pl.when(c != 0)
    def _acc():
        out_ref[0] += part


def _heads_kernel(bx_ref, mx_ref, w6_ref, b6_ref, w7_ref, b7_ref,
                  wcb_ref, bcb_ref, wt_ref, bt_ref, w1_ref, b1_ref,
                  box_ref, quad_ref):
    h = jnp.dot(bx_ref[...], w6_ref[...],
                preferred_element_type=jnp.float32) + b6_ref[...]
    h = jnp.maximum(h, 0.0)
    h = jnp.dot(h, w7_ref[...], preferred_element_type=jnp.float32) + b7_ref[...]
    h = jnp.maximum(h, 0.0)
    box_ref[...] = jnp.dot(h, wcb_ref[...],
                           preferred_element_type=jnp.float32) + bcb_ref[...]

    m = jnp.dot(mx_ref[...], wt_ref[...],
                preferred_element_type=jnp.float32) + bt_ref[...]
    m = jnp.maximum(m, 0.0)
    quad_ref[...] = jnp.dot(m, w1_ref[...],
                            preferred_element_type=jnp.float32) + b1_ref[...]


def kernel(images, w6_pad, b6_pad, w7_pad, b7_pad, wcb_pad, bcb_pad,
           wt_cat, bt_cat, w1_bd_pad, b1_cat_pad):
    b, ch, h, w = images.shape
    hblk, wblk = h // 8, w // 8
    inv = 1.0 / (ch * hblk * wblk)

    pool8 = pl.pallas_call(
        lambda x_ref, o_ref: _pool_kernel(x_ref, o_ref,
                                          hblk=hblk, wblk=wblk, inv=inv),
        out_shape=jax.ShapeDtypeStruct((b, 8, 8), jnp.float32),
        grid_spec=pltpu.PrefetchScalarGridSpec(
            num_scalar_prefetch=0,
            grid=(b, ch),
            in_specs=[pl.BlockSpec((1, 1, h, w), lambda i, c: (i, c, 0, 0))],
            out_specs=pl.BlockSpec((1, 8, 8), lambda i, c: (i, 0, 0)),
        ),
        compiler_params=pltpu.CompilerParams(
            dimension_semantics=("parallel", "arbitrary")),
    )(images)

    pool4 = pool8.reshape(b, 4, 2, 4, 2).mean(axis=(2, 4))

    roi_scale = 1.0 + 0.1 * jnp.arange(_ROIS_PER_IMG, dtype=jnp.float32)
    ch_scale = 1.0 + 0.01 * jnp.arange(_C, dtype=jnp.float32)
    n_roi = b * _ROIS_PER_IMG

    def synth(pooled, psize):
        f = (pooled[:, None, :, :, None]
             * roi_scale[None, :, None, None, None]
             * ch_scale[None, None, None, None, :])
        return f.reshape(n_roi, psize * psize * _C)

    box_x = synth(pool4, 4)
    mask_x = synth(pool8, 8).reshape(n_roi * 64, _C)

    box_rows = n_roi // 2
    mask_rows = (n_roi * 64) // 2
    box_out, quad = pl.pallas_call(
        _heads_kernel,
        out_shape=(jax.ShapeDtypeStruct((n_roi, 128), jnp.float32),
                   jax.ShapeDtypeStruct((n_roi * 64, 128), jnp.float32)),
        grid_spec=pltpu.PrefetchScalarGridSpec(
            num_scalar_prefetch=0,
            grid=(2,),
            in_specs=[
                pl.BlockSpec((box_rows, box_x.shape[1]), lambda i: (i, 0)),
                pl.BlockSpec((mask_rows, _C), lambda i: (i, 0)),
                pl.BlockSpec(w6_pad.shape, lambda i: (0, 0)),
                pl.BlockSpec(b6_pad.shape, lambda i: (0, 0)),
                pl.BlockSpec(w7_pad.shape, lambda i: (0, 0)),
                pl.BlockSpec(b7_pad.shape, lambda i: (0, 0)),
                pl.BlockSpec(wcb_pad.shape, lambda i: (0, 0)),
                pl.BlockSpec(bcb_pad.shape, lambda i: (0, 0)),
                pl.BlockSpec(wt_cat.shape, lambda i: (0, 0)),
                pl.BlockSpec(bt_cat.shape, lambda i: (0, 0)),
                pl.BlockSpec(w1_bd_pad.shape, lambda i: (0, 0)),
                pl.BlockSpec(b1_cat_pad.shape, lambda i: (0, 0)),
            ],
            out_specs=(pl.BlockSpec((box_rows, 128), lambda i: (i, 0)),
                       pl.BlockSpec((mask_rows, 128), lambda i: (i, 0))),
        ),
        compiler_params=pltpu.CompilerParams(
            dimension_semantics=("parallel",)),
    )(box_x, mask_x, w6_pad, b6_pad, w7_pad, b7_pad, wcb_pad, bcb_pad,
      wt_cat, bt_cat, w1_bd_pad, b1_cat_pad)

    cls_logits = box_out[:, :_NUM_CLASSES]
    bbox_deltas = box_out[:, _NUM_CLASSES:_BOX_OUT]

    quad = quad[:, :4 * _NUM_CLASSES]
    out = quad.reshape(n_roi, 8, 8, 2, 2, _NUM_CLASSES)
    out = jnp.transpose(out, (0, 1, 3, 2, 4, 5)).reshape(n_roi, 16, 16,
                                                         _NUM_CLASSES)
    mask_logits = jnp.transpose(out, (0, 3, 1, 2))
    return cls_logits, bbox_deltas, mask_logits


# speedup vs baseline: 4.0039x; 1.3267x over previous
import jax
import jax.numpy as jnp
from jax.experimental import pallas as pl
from jax.experimental.pallas import tpu as pltpu

_K = 4
_R = 8
_C = 32
_P4, _P8 = 4, 8


def _iota2(shape, dim):
    return jax.lax.broadcasted_iota(jnp.int32, shape, dim)


def _col_to_row(col):
    n = col.shape[0]
    eye = (_iota2((n, n), 0) == _iota2((n, n), 1)).astype(jnp.float32)
    return jnp.sum(col * eye, axis=0, keepdims=True)


def _fused_kernel(x_ref, w6_ref, b6_ref, w7_ref, b7_ref, wcb_ref, bcb_ref,
                  wt_ref, bt_ref, w1_ref, b1_ref,
                  cls_ref, bbox_ref, quad_ref, acc,
                  *, hblk, wblk, nch):
    c = pl.program_id(1)
    x = x_ref[0, 0]
    h, w = x.shape
    inv = 1.0 / (nch * hblk * wblk)

    s_left = (_iota2((_P8, h), 1) // hblk == _iota2((_P8, h), 0)
              ).astype(jnp.float32)
    s_right = (_iota2((w, _P8), 0) // wblk == _iota2((w, _P8), 1)
               ).astype(jnp.float32)
    rows = jnp.dot(s_left, x, preferred_element_type=jnp.float32)
    part = jnp.dot(rows, s_right, preferred_element_type=jnp.float32) * inv

    @pl.when(c == 0)
    def _init():
        acc[...] = part

    @pl.when(c != 0)
    def _accum():
        acc[...] += part

    @pl.when(c == nch - 1)
    def _heads():
        pool8 = acc[...]

        e64 = (_iota2((64, _P8), 1) == _iota2((64, _P8), 0) // _P8
               ).astype(jnp.float32)
        m64 = (_iota2((64, _P8), 1) == _iota2((64, _P8), 0) % _P8
               ).astype(jnp.float32)
        p64 = jnp.sum(jnp.dot(e64, pool8, preferred_element_type=jnp.float32)
                      * m64, axis=1, keepdims=True)

        ki, ni = _iota2((16, 64), 0), _iota2((16, 64), 1)
        q4 = (((ni // _P8) // 2 == ki // _P4)
              & ((ni % _P8) // 2 == ki % _P4)).astype(jnp.float32) * 0.25
        pool4 = jnp.dot(q4, p64, preferred_element_type=jnp.float32)

        ki, mi = _iota2((16, 16 * _C), 0), _iota2((16, 16 * _C), 1)
        sel = jnp.where(mi // _C == ki,
                        1.0 + 0.01 * (mi % _C).astype(jnp.float32), 0.0)
        w6f = jnp.dot(sel, w6_ref[...],
                      preferred_element_type=jnp.float32)
        v = jnp.dot(_col_to_row(pool4), w6f,
                    preferred_element_type=jnp.float32)
        roi = 1.0 + 0.1 * _iota2((_R, 1), 0).astype(jnp.float32)
        hb = jnp.maximum(roi * v + b6_ref[...], 0.0)
        hb = jnp.maximum(jnp.dot(hb, w7_ref[...],
                                 preferred_element_type=jnp.float32)
                         + b7_ref[...], 0.0)
        box = jnp.dot(hb, wcb_ref[...],
                      preferred_element_type=jnp.float32) + bcb_ref[...]
        cls_ref[...] = box[:, :_K]
        bbox_ref[...] = box[:, _K:5 * _K]

        mi, ni = _iota2((_R * 64, 64), 0), _iota2((_R * 64, 64), 1)
        rmat = jnp.where(mi % 64 == ni,
                         1.0 + 0.1 * (mi // 64).astype(jnp.float32), 0.0)
        s = jnp.dot(rmat, p64, preferred_element_type=jnp.float32)
        ch_row = 1.0 + 0.01 * _iota2((1, _C), 1).astype(jnp.float32)
        u = jnp.dot(ch_row, wt_ref[...],
                    preferred_element_type=jnp.float32)
        hm = jnp.maximum(s * u + bt_ref[...], 0.0)
        quad_ref[...] = jnp.dot(hm, w1_ref[...],
                                preferred_element_type=jnp.float32) + b1_ref[...]


def kernel(images, w6_pad, b6_pad, w7_pad, b7_pad, wcb_pad, bcb_pad,
           wt_cat, bt_cat, w1_bd_pad, b1_cat_pad):
    b, nch, h, w = images.shape
    hblk, wblk = h // _P8, w // _P8
    n_roi = b * _R
    mask_rows = _R * _P8 * _P8

    import functools
    body = functools.partial(_fused_kernel, hblk=hblk, wblk=wblk, nch=nch)

    bcast = lambda i, c: (0, 0)
    cls, bbox, quad = pl.pallas_call(
        body,
        out_shape=(jax.ShapeDtypeStruct((n_roi, _K), jnp.float32),
                   jax.ShapeDtypeStruct((n_roi, 4 * _K), jnp.float32),
                   jax.ShapeDtypeStruct((b * mask_rows, 128), jnp.float32)),
        grid_spec=pltpu.PrefetchScalarGridSpec(
            num_scalar_prefetch=0,
            grid=(b, nch),
            in_specs=[
                pl.BlockSpec((1, 1, h, w), lambda i, c: (i, c, 0, 0)),
                pl.BlockSpec(w6_pad.shape, bcast),
                pl.BlockSpec(b6_pad.shape, bcast),
                pl.BlockSpec(w7_pad.shape, bcast),
                pl.BlockSpec(b7_pad.shape, bcast),
                pl.BlockSpec(wcb_pad.shape, bcast),
                pl.BlockSpec(bcb_pad.shape, bcast),
                pl.BlockSpec(wt_cat.shape, bcast),
                pl.BlockSpec(bt_cat.shape, bcast),
                pl.BlockSpec(w1_bd_pad.shape, bcast),
                pl.BlockSpec(b1_cat_pad.shape, bcast),
            ],
            out_specs=(pl.BlockSpec((_R, _K), lambda i, c: (i, 0)),
                       pl.BlockSpec((_R, 4 * _K), lambda i, c: (i, 0)),
                       pl.BlockSpec((mask_rows, 128), lambda i, c: (i, 0))),
            scratch_shapes=[pltpu.VMEM((_P8, _P8), jnp.float32)],
        ),
        compiler_params=pltpu.CompilerParams(
            dimension_semantics=("parallel", "arbitrary")),
    )(images, w6_pad, b6_pad, w7_pad, b7_pad, wcb_pad, bcb_pad,
      wt_cat, bt_cat, w1_bd_pad, b1_cat_pad)

    out = quad[:, :4 * _K].reshape(n_roi, _P8, _P8, 2, 2, _K)
    out = jnp.transpose(out, (0, 1, 3, 2, 4, 5)).reshape(n_roi, 2 * _P8,
                                                         2 * _P8, _K)
    mask_logits = jnp.transpose(out, (0, 3, 1, 2))
    return cls, bbox, mask_logits
